# baseline (speedup 1.0000x reference)
import functools
import jax
import jax.numpy as jnp
from jax import lax
from jax.experimental import pallas as pl
from jax.experimental.pallas import tpu as pltpu
from jax.experimental.pallas import tpu_sc as plsc

NC, NS = 2, 16
NW = NC * NS


def _sc_gather_body(idx_hbm, cv_hbm, out_hbm, idx_v, rows_v, sem):
    wid = lax.axis_index("s") * NC + lax.axis_index("c")

    @pl.when(wid < 4)
    def _():
        base = wid * 8
        pltpu.sync_copy(idx_hbm.at[pl.ds(base, 8)], idx_v)
        pltpu.async_copy(cv_hbm.at[idx_v], rows_v, sem).wait()
        pltpu.sync_copy(rows_v, out_hbm.at[pl.ds(base, 8)])


def _sc_gather(idx, control_vectors):
    B = idx.shape[0]
    n, E = control_vectors.shape
    mesh = plsc.VectorSubcoreMesh(core_axis_name="c", subcore_axis_name="s")
    return pl.kernel(
        _sc_gather_body,
        out_type=jax.ShapeDtypeStruct((B, E), jnp.float32),
        mesh=mesh,
        scratch_types=[
            pltpu.VMEM((8,), jnp.int32),
            pltpu.VMEM((8, E), jnp.float32),
            pltpu.SemaphoreType.DMA,
        ],
    )(idx, control_vectors)


def _tc_add_body(h_ref, a_ref, o_ref):
    o_ref[...] = h_ref[...] + a_ref[0]


def kernel(hidden_states, affective_state_indices, control_vectors):
    B, S, E = hidden_states.shape
    n = control_vectors.shape[0]
    idx = jnp.clip(affective_state_indices.astype(jnp.int32), 0, n - 1)
    adj = _sc_gather(idx, control_vectors).reshape(B, 1, E)
    BS = 2048
    grid = (B, S // BS)

    def h_map(b, s):
        return (b, s, 0)

    return pl.pallas_call(
        _tc_add_body,
        grid=grid,
        in_specs=[
            pl.BlockSpec((1, BS, E), h_map),
            pl.BlockSpec((1, 1, E), lambda b, s: (b, 0, 0)),
        ],
        out_specs=pl.BlockSpec((1, BS, E), h_map),
        out_shape=jax.ShapeDtypeStruct((B, S, E), hidden_states.dtype),
    )(hidden_states, adj)
